# Initial kernel scaffold; baseline (speedup 1.0000x reference)
import jax
import jax.numpy as jnp
from jax import lax
from jax.experimental import pallas as pl
from jax.experimental.pallas import tpu as pltpu
from jax.experimental.pallas import tpu_sc as plsc

EMBED = 64
LANES = 16
NC, NS = 2, 16
NW = NC * NS

ROWS = 16384 * 200
ROWS_PER_TILE = ROWS // NW
CHUNK = 1024
NCHUNK = ROWS_PER_TILE // CHUNK
GATHER = 128
NGATHER = CHUNK // GATHER


def _body(x_hbm, table_hbm, out_hbm, xv, idxv, rows, sem):
    wid = lax.axis_index("s") * NC + lax.axis_index("c")
    base = wid * ROWS_PER_TILE

    def chunk_body(ci, carry):
        rowbase = base + ci * CHUNK
        pltpu.sync_copy(x_hbm.at[pl.ds(rowbase, CHUNK)], xv)
        for i in range(CHUNK // LANES):
            j, k = divmod(i, GATHER // LANES)
            v = xv[pl.ds(i * LANES, LANES)]
            idxv[j, pl.ds(k * LANES, LANES)] = lax.rem(v, 7)
        copies = [
            pltpu.async_copy(
                table_hbm.at[idxv.at[j]], rows.at[pl.ds(j * GATHER, GATHER)], sem
            )
            for j in range(NGATHER)
        ]
        for c in copies:
            c.wait()
        pltpu.sync_copy(rows, out_hbm.at[pl.ds(rowbase, CHUNK)])
        return carry

    lax.fori_loop(0, NCHUNK, chunk_body, 0)


def kernel(x, table):
    x_flat = x.reshape(ROWS).astype(jnp.int32)
    mesh = plsc.VectorSubcoreMesh(core_axis_name="c", subcore_axis_name="s")
    out = pl.kernel(
        _body,
        out_type=jax.ShapeDtypeStruct((ROWS, EMBED), jnp.float32),
        mesh=mesh,
        scratch_types=[
            pltpu.VMEM((CHUNK,), jnp.int32),
            pltpu.VMEM((NGATHER, GATHER), jnp.int32),
            pltpu.VMEM((CHUNK, EMBED), jnp.float32),
            pltpu.SemaphoreType.DMA,
        ],
    )(x_flat, table)
    return out.reshape(x.shape[0], x.shape[1], EMBED)


# speedup vs baseline: 2.3504x; 2.3504x over previous
import jax
import jax.numpy as jnp
from jax import lax
from jax.experimental import pallas as pl
from jax.experimental.pallas import tpu as pltpu
from jax.experimental.pallas import tpu_sc as plsc

EMBED = 64
LANES = 16
NC, NS = 2, 16
NW = NC * NS

ROWS = 16384 * 200
PAIRS = ROWS // 2
CHUNK = 512
NCHUNK = PAIRS // (NW * CHUNK)
GATHER = 128
NGATHER = CHUNK // GATHER


def _body(xt_hbm, ptable_hbm, out_hbm, xev, xov, pidx, rows, sem):
    wid = lax.axis_index("s") * NC + lax.axis_index("c")
    base = wid * CHUNK

    def chunk_body(ci, carry):
        pairbase = base + ci * (NW * CHUNK)
        pltpu.sync_copy(xt_hbm.at[0, pl.ds(pairbase, CHUNK)], xev)
        pltpu.sync_copy(xt_hbm.at[1, pl.ds(pairbase, CHUNK)], xov)
        for i in range(CHUNK // LANES):
            j, k = divmod(i, GATHER // LANES)
            ev = xev[pl.ds(i * LANES, LANES)]
            od = xov[pl.ds(i * LANES, LANES)]
            pidx[j, pl.ds(k * LANES, LANES)] = (
                lax.rem(ev, 7) * 7 + lax.rem(od, 7)
            )
        copies = [
            pltpu.async_copy(
                ptable_hbm.at[pidx.at[j]], rows.at[pl.ds(j * GATHER, GATHER)], sem
            )
            for j in range(NGATHER)
        ]
        for c in copies:
            c.wait()
        pltpu.sync_copy(rows, out_hbm.at[pl.ds(pairbase, CHUNK)])
        return carry

    lax.fori_loop(0, NCHUNK, chunk_body, 0)


def kernel(x, table):
    xt = x.reshape(PAIRS, 2).astype(jnp.int32).T
    ptable = jnp.concatenate(
        [
            jnp.broadcast_to(table[:, None, :], (7, 7, EMBED)),
            jnp.broadcast_to(table[None, :, :], (7, 7, EMBED)),
        ],
        axis=-1,
    ).reshape(49, 2 * EMBED)
    mesh = plsc.VectorSubcoreMesh(core_axis_name="c", subcore_axis_name="s")
    out = pl.kernel(
        _body,
        out_type=jax.ShapeDtypeStruct((PAIRS, 2 * EMBED), jnp.float32),
        mesh=mesh,
        scratch_types=[
            pltpu.VMEM((CHUNK,), jnp.int32),
            pltpu.VMEM((CHUNK,), jnp.int32),
            pltpu.VMEM((NGATHER, GATHER), jnp.int32),
            pltpu.VMEM((CHUNK, 2 * EMBED), jnp.float32),
            pltpu.SemaphoreType.DMA,
        ],
    )(xt, ptable)
    return out.reshape(x.shape[0], x.shape[1], EMBED)
